# baseline (speedup 1.0000x reference)
import functools

import jax
import jax.numpy as jnp
from jax import lax
from jax.experimental import pallas as pl
from jax.experimental.pallas import tpu as pltpu
from jax.experimental.pallas import tpu_sc as plsc

SEQ = 200
DIM = 32
NC = 2
NS = 16
NW = NC * NS

CHUNK = 1600
GROUP = 64
NGRP = CHUNK // GROUP
NB = 5
LANES = 16


def _emb_kernel(n_rows, idx_hbm, tok_hbm, pos_hbm, out_hbm,
                pos_v, idx_v, qidx_v, bufs, outbufs, gsems, wsems):
    rows_per_w = n_rows // NW
    n_chunks = rows_per_w // CHUNK

    wid = lax.axis_index("s") * NC + lax.axis_index("c")
    base0 = wid * rows_per_w

    pltpu.sync_copy(pos_hbm, pos_v)

    def fire_gather(go, slot):
        pltpu.async_copy(tok_hbm.at[qidx_v.at[pl.ds(go * GROUP, GROUP)]],
                         bufs[slot], gsems[slot])

    def wait_gather(go, slot):
        pltpu.make_async_copy(tok_hbm.at[qidx_v.at[pl.ds(go * GROUP, GROUP)]],
                              bufs[slot], gsems[slot]).wait()

    def out_rows(c, go):
        r = (base0 + c * CHUNK) * DIM // 128 + go * (GROUP * DIM // 128)
        return pl.multiple_of(r, 16)

    def fire_wb(c, go, slot):
        pltpu.async_copy(outbufs[slot],
                         out_hbm.at[pl.ds(out_rows(c, go), GROUP * DIM // 128)],
                         wsems[slot])

    def wait_wb(c, go, slot):
        pltpu.make_async_copy(
            outbufs[slot],
            out_hbm.at[pl.ds(out_rows(c, go), GROUP * DIM // 128)],
            wsems[slot]).wait()

    def select_group(c, go, slot):
        buf = bufs[slot]
        outb = outbufs[slot]
        grow0 = go * GROUP

        def blk_body(blk, carry):
            r0 = blk * LANES
            idx_vec = idx_v[pl.ds(grow0 + r0, LANES)]
            for l in range(LANES):
                tok = idx_vec[l]
                off = (tok & 3) << 5
                prow = lax.rem(grow0 + (r0 + l), SEQ)
                v0 = buf[r0 + l, pl.ds(off, 16)] + pos_v[prow, pl.ds(0, 16)]
                v1 = (buf[r0 + l, pl.ds(off + 16, 16)]
                      + pos_v[prow, pl.ds(16, 16)])
                orow = (r0 + l) // 4
                ocol = ((r0 + l) % 4) * DIM
                outb[orow, pl.ds(ocol, 16)] = v0
                outb[orow, pl.ds(ocol + 16, 16)] = v1
            return carry

        lax.fori_loop(0, GROUP // LANES, blk_body, 0)

    def chunk_body(c, carry):
        cbase = base0 + c * CHUNK
        pltpu.sync_copy(idx_hbm.at[pl.ds(cbase, CHUNK)], idx_v)

        @plsc.parallel_loop(0, CHUNK // LANES, unroll=8)
        def _q(j):
            qidx_v[pl.ds(j * LANES, LANES)] = lax.shift_right_logical(
                idx_v[pl.ds(j * LANES, LANES)], 2)

        for slot in range(NB - 1):
            fire_gather(slot, slot)

        def ring_body(i, c2):
            for b in range(NB):
                go = i * NB + b

                @pl.when(go + NB - 1 < NGRP)
                def _fire():
                    fire_gather(go + NB - 1, (b + NB - 1) % NB)

                wait_gather(go, b)

                @pl.when(go >= NB)
                def _wb_done():
                    wait_wb(c, go - NB, b)

                @pl.when(jnp.logical_and(go < NB, c >= 1))
                def _wb_done_prev():
                    wait_wb(c - 1, go + NGRP - NB, b)

                select_group(c, go, b)
                fire_wb(c, go, b)
            return c2

        lax.fori_loop(0, NGRP // NB, ring_body, 0)
        return carry

    lax.fori_loop(0, n_chunks, chunk_body, 0)
    for b in range(NB):
        wait_wb(n_chunks - 1, NGRP - NB + b, b)


def kernel(inputs, token_table, pos_table):
    b, s = inputs.shape
    n_rows = b * s
    assert s == SEQ and token_table.shape[1] == DIM
    assert n_rows % (NW * CHUNK) == 0 and token_table.shape[0] % 4 == 0

    idx = inputs.reshape(n_rows).astype(jnp.int32)
    tok_128 = token_table.reshape(token_table.size // 128, 128)

    mesh = plsc.VectorSubcoreMesh(core_axis_name="c", subcore_axis_name="s")
    k = functools.partial(
        pl.kernel,
        mesh=mesh,
        out_type=jax.ShapeDtypeStruct((n_rows * DIM // 128, 128), jnp.float32),
        scratch_types=[
            pltpu.VMEM((SEQ, DIM), jnp.float32),
            pltpu.VMEM((CHUNK,), jnp.int32),
            pltpu.VMEM((CHUNK,), jnp.int32),
            [pltpu.VMEM((GROUP, 128), jnp.float32)] * NB,
            [pltpu.VMEM((GROUP * DIM // 128, 128), jnp.float32)] * NB,
            [pltpu.SemaphoreType.DMA] * NB,
            [pltpu.SemaphoreType.DMA] * NB,
        ],
    )(functools.partial(_emb_kernel, n_rows))

    out = k(idx, tok_128, pos_table)
    return out.reshape(b, s, DIM)


# speedup vs baseline: 1.0625x; 1.0625x over previous
import functools

import jax
import jax.numpy as jnp
from jax import lax
from jax.experimental import pallas as pl
from jax.experimental.pallas import tpu as pltpu
from jax.experimental.pallas import tpu_sc as plsc

SEQ = 200
DIM = 32
NC = 2
NS = 16
NW = NC * NS

SPC = 4
CHUNK = SPC * SEQ
LANES = 16
NBLK = 13


def _emb_kernel(n_seq, idx_hbm, tok_hbm, pos_hbm, out_hbm,
                pos_v, idx_v, qidx_v, g0, g1, o0, o1, gs0, gs1, ws0, ws1):
    seq_per_w = n_seq // NW
    n_chunks = seq_per_w // SPC

    wid = lax.axis_index("s") * NC + lax.axis_index("c")
    seq0 = wid * seq_per_w
    sets = ((g0, o0, gs0, ws0), (g1, o1, gs1, ws1))

    pltpu.sync_copy(pos_hbm, pos_v)

    def fire_gather(sl, gbuf, gsem):
        pltpu.async_copy(tok_hbm.at[qidx_v.at[pl.ds(sl * SEQ, SEQ)]],
                         gbuf, gsem)

    def wait_gather(sl, gbuf, gsem):
        pltpu.make_async_copy(tok_hbm.at[qidx_v.at[pl.ds(sl * SEQ, SEQ)]],
                              gbuf, gsem).wait()

    def fire_wb(s_abs, obuf, wsem):
        pltpu.async_copy(obuf, out_hbm.at[s_abs], wsem)

    def wait_wb(s_abs, obuf, wsem):
        pltpu.make_async_copy(obuf, out_hbm.at[s_abs], wsem).wait()

    def select_seq(sl, gbuf, obuf):
        soff = sl * SEQ

        def blk_body(blk, carry):
            p0 = lax.min(blk * LANES, SEQ - LANES)
            idx_vec = idx_v[pl.ds(soff + p0, LANES)]
            for l in range(LANES):
                off = (idx_vec[l] & 3) << 5
                p = p0 + l
                v0 = gbuf[p, pl.ds(off, 16)] + pos_v[p, pl.ds(0, 16)]
                v1 = gbuf[p, pl.ds(off + 16, 16)] + pos_v[p, pl.ds(16, 16)]
                obuf[p, pl.ds(0, 16)] = v0
                obuf[p, pl.ds(16, 16)] = v1
            return carry

        lax.fori_loop(0, NBLK, blk_body, 0)

    def chunk_body(c, carry):
        sbase = seq0 + c * SPC
        pltpu.sync_copy(idx_hbm.at[pl.ds(sbase * SEQ, CHUNK)], idx_v)

        @plsc.parallel_loop(0, CHUNK // LANES, unroll=8)
        def _q(j):
            qidx_v[pl.ds(j * LANES, LANES)] = lax.shift_right_logical(
                idx_v[pl.ds(j * LANES, LANES)], 2)

        fire_gather(0, g0, gs0)

        def pair_body(i, c2):
            for b in (0, 1):
                gbuf, obuf, gsem, wsem = sets[b]
                ngbuf, _, ngsem, _ = sets[1 - b]
                sl = 2 * i + b

                @pl.when(sl + 1 < SPC)
                def _fire():
                    fire_gather(sl + 1, ngbuf, ngsem)

                wait_gather(sl, gbuf, gsem)

                @pl.when(jnp.logical_or(sl >= 2, c >= 1))
                def _wb_done():
                    wait_wb(sbase + sl - 2, obuf, wsem)

                select_seq(sl, gbuf, obuf)
                fire_wb(sbase + sl, obuf, wsem)
            return c2

        lax.fori_loop(0, SPC // 2, pair_body, 0)
        return carry

    lax.fori_loop(0, n_chunks, chunk_body, 0)
    last = seq0 + n_chunks * SPC
    wait_wb(last - 2, o0, ws0)
    wait_wb(last - 1, o1, ws1)


def kernel(inputs, token_table, pos_table):
    b, s = inputs.shape
    n_rows = b * s
    assert s == SEQ and token_table.shape[1] == DIM
    assert b % (NW * SPC) == 0 and token_table.shape[0] % 4 == 0

    idx = inputs.reshape(n_rows).astype(jnp.int32)
    tok_4 = token_table.reshape(token_table.size // 128, 128)

    mesh = plsc.VectorSubcoreMesh(core_axis_name="c", subcore_axis_name="s")
    k = functools.partial(
        pl.kernel,
        mesh=mesh,
        out_type=jax.ShapeDtypeStruct((b, s, DIM), jnp.float32),
        scratch_types=[
            pltpu.VMEM((SEQ, DIM), jnp.float32),
            pltpu.VMEM((CHUNK,), jnp.int32),
            pltpu.VMEM((CHUNK,), jnp.int32),
            pltpu.VMEM((SEQ, 128), jnp.float32),
            pltpu.VMEM((SEQ, 128), jnp.float32),
            pltpu.VMEM((SEQ, DIM), jnp.float32),
            pltpu.VMEM((SEQ, DIM), jnp.float32),
            pltpu.SemaphoreType.DMA,
            pltpu.SemaphoreType.DMA,
            pltpu.SemaphoreType.DMA,
            pltpu.SemaphoreType.DMA,
        ],
    )(functools.partial(_emb_kernel, b))

    return k(idx, tok_4, pos_table)


# speedup vs baseline: 1.0682x; 1.0054x over previous
import functools

import jax
import jax.numpy as jnp
from jax import lax
from jax.experimental import pallas as pl
from jax.experimental.pallas import tpu as pltpu
from jax.experimental.pallas import tpu_sc as plsc

SEQ = 200
DIM = 32
NC = 2
NS = 16
NW = NC * NS

LANES = 16
GB = 64
NRING = 4
PP = 4
SUBS = 16


def _emb_kernel(n_seq, idx_hbm, tok_hbm, pos_hbm, out_hbm,
                pos_v, idx_v, qidx_v, gbufs, obufs, gsems, wsems):
    seq_per_w = n_seq // NW
    halves = seq_per_w // GB

    wid = lax.axis_index("s") * NC + lax.axis_index("c")
    b0w = wid * seq_per_w

    pltpu.sync_copy(idx_hbm.at[pl.ds(b0w * SEQ, seq_per_w * SEQ)], idx_v)
    pltpu.sync_copy(pos_hbm, pos_v)

    iota200 = lax.iota(jnp.int32, LANES) * SEQ
    iota16 = lax.iota(jnp.int32, LANES)

    def tokens_at(p, bh, blk):
        base = (bh * GB + blk * LANES) * SEQ + p
        return plsc.load_gather(idx_v, [base + iota200])

    def build_qidx(p, bh, slot):
        for blk in range(GB // LANES):
            toks = tokens_at(p, bh, blk)
            qidx_v[slot, pl.ds(blk * LANES, LANES)] = (
                lax.shift_right_logical(toks, 2))

    def fire_gather(slot):
        pltpu.async_copy(tok_hbm.at[qidx_v.at[slot]], gbufs[slot],
                         gsems[slot])

    def wait_gather(slot):
        pltpu.make_async_copy(tok_hbm.at[qidx_v.at[slot]], gbufs[slot],
                              gsems[slot]).wait()

    def wb_dst(p0, ob):
        b_off = pl.multiple_of(b0w, seq_per_w)
        return out_hbm.at[pl.ds(p0, PP), :, pl.ds(b_off, seq_per_w)]

    def fire_wb(p0, ob):
        pltpu.async_copy(obufs[ob], wb_dst(p0, ob), wsems[ob])

    def wait_wb(p0, ob):
        pltpu.make_async_copy(obufs[ob], wb_dst(p0, ob), wsems[ob]).wait()

    def select_group(p, p_loc, bh, slot, ob):
        gbuf = gbufs[slot]
        obuf = obufs[ob]
        pos0 = pos_v[p, pl.ds(0, 16)]
        pos1 = pos_v[p, pl.ds(16, 16)]
        pvec = jnp.full((LANES,), p_loc, jnp.int32)

        def blk_body(blk, carry):
            toks = tokens_at(p, bh, blk)
            for l in range(LANES):
                off = (toks[l] & 3) << 5
                row = blk * LANES + l
                v0 = gbuf[row, pl.ds(off, 16)] + pos0
                v1 = gbuf[row, pl.ds(off + 16, 16)] + pos1
                bvec = jnp.full((LANES,), bh * GB + row, jnp.int32)
                plsc.store_scatter(obuf, [pvec, iota16, bvec], v0)
                plsc.store_scatter(obuf, [pvec, iota16 + 16, bvec], v1)
            return carry

        lax.fori_loop(0, GB // LANES, blk_body, 0)

    for g in range(NRING - 1):
        build_qidx(g // halves, g % halves, g % NRING)
        fire_gather(g % NRING)

    n_steps = SEQ * halves // SUBS

    def step_body(i, carry):
        for sub in range(SUBS):
            g = i * SUBS + sub
            p = i * (SUBS // halves) + sub // halves
            bh = sub % halves
            slot = sub % NRING
            ob = sub // (SUBS // 2)
            p_loc = (sub // halves) % PP

            @pl.when(g + NRING - 1 < SEQ * halves)
            def _fire():
                gn = g + NRING - 1
                build_qidx(gn // halves, gn % halves, (sub + NRING - 1) % NRING)
                fire_gather((sub + NRING - 1) % NRING)

            wait_gather(slot)

            if sub % (SUBS // 2) == 0:
                @pl.when(i >= 1)
                def _wb_done():
                    wait_wb((i - 1) * (SUBS // halves) + ob * PP, ob)

            select_group(p, p_loc, bh, slot, ob)

            if sub % (SUBS // 2) == (SUBS // 2) - 1:
                fire_wb(i * (SUBS // halves) + ob * PP, ob)
        return carry

    lax.fori_loop(0, n_steps, step_body, 0)
    wait_wb((n_steps - 1) * (SUBS // halves), 0)
    wait_wb((n_steps - 1) * (SUBS // halves) + PP, 1)


def kernel(inputs, token_table, pos_table):
    b, s = inputs.shape
    n_rows = b * s
    assert s == SEQ and token_table.shape[1] == DIM
    assert b % (NW * GB) == 0 and token_table.shape[0] % 4 == 0

    idx = inputs.reshape(n_rows).astype(jnp.int32)
    tok_4 = token_table.reshape(token_table.size // 128, 128)

    mesh = plsc.VectorSubcoreMesh(core_axis_name="c", subcore_axis_name="s")
    k = functools.partial(
        pl.kernel,
        mesh=mesh,
        compiler_params=pltpu.CompilerParams(needs_layout_passes=False),
        out_type=jax.ShapeDtypeStruct((SEQ, DIM, b), jnp.float32),
        scratch_types=[
            pltpu.VMEM((SEQ, DIM), jnp.float32),
            pltpu.VMEM((b // NW * SEQ,), jnp.int32),
            pltpu.VMEM((NRING, GB), jnp.int32),
            [pltpu.VMEM((GB, 128), jnp.float32)] * NRING,
            [pltpu.VMEM((PP, DIM, b // NW), jnp.float32)] * 2,
            [pltpu.SemaphoreType.DMA] * NRING,
            [pltpu.SemaphoreType.DMA] * 2,
        ],
    )(functools.partial(_emb_kernel, b))

    out = k(idx, tok_4, pos_table)
    return out.transpose(2, 0, 1)
